# baseline (speedup 1.0000x reference)
import functools

import numpy as np
import jax
import jax.numpy as jnp
from jax import lax
from jax.experimental import pallas as pl
from jax.experimental.pallas import tpu as pltpu
from jax.experimental.pallas import tpu_sc as plsc

_PS = 16
_H = 512
_W = 512
_HP = _H // _PS
_WP = _W // _PS
_TOTAL = _HP * _WP
_NUM_MASK = int(0.75 * _TOTAL)
_MASK_VALUE = 0.5
_BLOCK_ROWS = 4096

_MASK_BITS_HEX = (
    "bfbe67fd4f3fa775bcfdfe7dffefe7bbf0f9ff37fadbfefe6c7bfffaff4b5b6f"
    "fdabf03bd7ffbd7ffdeffa7f5bbe7fefe8e74efffffff7feeefffbf7f5f3b57d"
    "f9baefd79ff8febdf7f1affaceed6bb4fdcfdc3e677fbcbb4fbbf4cad97fb7ef"
    "efffffd49e3ecffdff9fe299ff5b5e9f0a65d66b75effbeefd76bdefe3dfeffd"
)


def _full_mask() -> np.ndarray:
    val = int(_MASK_BITS_HEX, 16)
    patch_mask = np.array([(val >> i) & 1 for i in range(_TOTAL)], dtype=bool)
    grid2d = patch_mask.reshape(_HP, _WP)
    return np.repeat(np.repeat(grid2d, _PS, axis=0), _PS, axis=1)


_MASK_BLOCK = np.tile(_full_mask(), (_BLOCK_ROWS // _H, 1)).astype(np.float32)


def _select_body(m_ref, x_ref, o_ref):
    o_ref[...] = jnp.where(m_ref[...] != 0.0, _MASK_VALUE, x_ref[...])


def _kernel_tc(x):
    B, C, H, W = x.shape
    rows = B * C * H
    xr = x.reshape(rows, W)
    mask = jnp.asarray(_MASK_BLOCK)
    out = pl.pallas_call(
        _select_body,
        grid=(rows // _BLOCK_ROWS,),
        in_specs=[
            pl.BlockSpec((_BLOCK_ROWS, W), lambda i: (0, 0)),
            pl.BlockSpec((_BLOCK_ROWS, W), lambda i: (i, 0)),
        ],
        out_specs=pl.BlockSpec((_BLOCK_ROWS, W), lambda i: (i, 0)),
        out_shape=jax.ShapeDtypeStruct((rows, W), x.dtype),
        compiler_params=pltpu.CompilerParams(
            dimension_semantics=("parallel",),
        ),
    )(mask, xr)
    return out.reshape(B, C, H, W)


_L = 16
_NW = 32
_ROWS = 64 * 3 * _H
_RPW = _ROWS // _NW
_CHUNK = _PS
_NCHUNK = _RPW // _CHUNK


def _mask_row_table() -> np.ndarray:
    full = _full_mask()
    return full[::_PS, :].astype(np.float32)


_MASK_ROWS = _mask_row_table()


def _sc_body(x_hbm, mrow_hbm, out_hbm, buf, mtab):
    wid = lax.axis_index("s") * 2 + lax.axis_index("c")
    base = wid * _RPW
    pltpu.sync_copy(mrow_hbm, mtab)

    def chunk(ci, carry):
        row0 = base + ci * _CHUNK
        pltpu.sync_copy(x_hbm.at[pl.ds(row0, _CHUNK)], buf)
        pr = lax.rem(ci, _HP)
        mvals = [mtab[pr, pl.ds(c * _L, _L)] for c in range(_W // _L)]
        for r in range(_CHUNK):
            for c in range(_W // _L):
                sl = pl.ds(c * _L, _L)
                buf[r, sl] = jnp.where(mvals[c] != 0.0, _MASK_VALUE, buf[r, sl])
        pltpu.sync_copy(buf, out_hbm.at[pl.ds(row0, _CHUNK)])
        return carry

    lax.fori_loop(0, _NCHUNK, chunk, 0)


def _kernel_sc(x):
    B, C, H, W = x.shape
    xr = x.reshape(B * C * H, W)
    mrows = jnp.asarray(_MASK_ROWS)
    mesh = plsc.VectorSubcoreMesh(core_axis_name="c", subcore_axis_name="s")
    run = functools.partial(
        pl.kernel,
        mesh=mesh,
        out_type=jax.ShapeDtypeStruct((B * C * H, W), x.dtype),
        scratch_types=[
            pltpu.VMEM((_CHUNK, W), jnp.float32),
            pltpu.VMEM((_HP, W), jnp.float32),
        ],
    )(_sc_body)
    out = run(xr, mrows)
    return out.reshape(B, C, H, W)


kernel = _kernel_sc


# speedup vs baseline: 1.3279x; 1.3279x over previous
import functools

import numpy as np
import jax
import jax.numpy as jnp
from jax import lax
from jax.experimental import pallas as pl
from jax.experimental.pallas import tpu as pltpu
from jax.experimental.pallas import tpu_sc as plsc

_PS = 16
_H = 512
_W = 512
_HP = _H // _PS
_WP = _W // _PS
_TOTAL = _HP * _WP
_NUM_MASK = int(0.75 * _TOTAL)
_MASK_VALUE = 0.5
_BLOCK_ROWS = 4096

_MASK_BITS_HEX = (
    "bfbe67fd4f3fa775bcfdfe7dffefe7bbf0f9ff37fadbfefe6c7bfffaff4b5b6f"
    "fdabf03bd7ffbd7ffdeffa7f5bbe7fefe8e74efffffff7feeefffbf7f5f3b57d"
    "f9baefd79ff8febdf7f1affaceed6bb4fdcfdc3e677fbcbb4fbbf4cad97fb7ef"
    "efffffd49e3ecffdff9fe299ff5b5e9f0a65d66b75effbeefd76bdefe3dfeffd"
)


def _full_mask() -> np.ndarray:
    val = int(_MASK_BITS_HEX, 16)
    patch_mask = np.array([(val >> i) & 1 for i in range(_TOTAL)], dtype=bool)
    grid2d = patch_mask.reshape(_HP, _WP)
    return np.repeat(np.repeat(grid2d, _PS, axis=0), _PS, axis=1)


_MASK_BLOCK = np.tile(_full_mask(), (_BLOCK_ROWS // _H, 1)).astype(np.float32)


def _select_body(m_ref, x_ref, o_ref):
    o_ref[...] = jnp.where(m_ref[...] != 0.0, _MASK_VALUE, x_ref[...])


def _kernel_tc(x):
    B, C, H, W = x.shape
    rows = B * C * H
    xr = x.reshape(rows, W)
    mask = jnp.asarray(_MASK_BLOCK)
    out = pl.pallas_call(
        _select_body,
        grid=(rows // _BLOCK_ROWS,),
        in_specs=[
            pl.BlockSpec((_BLOCK_ROWS, W), lambda i: (0, 0)),
            pl.BlockSpec((_BLOCK_ROWS, W), lambda i: (i, 0)),
        ],
        out_specs=pl.BlockSpec((_BLOCK_ROWS, W), lambda i: (i, 0)),
        out_shape=jax.ShapeDtypeStruct((rows, W), x.dtype),
        compiler_params=pltpu.CompilerParams(
            dimension_semantics=("parallel",),
        ),
    )(mask, xr)
    return out.reshape(B, C, H, W)


_L = 16
_NW = 32
_ROWS = 64 * 3 * _H
_RPW = _ROWS // _NW
_CHUNK = _PS
_NCHUNK = _RPW // _CHUNK


def _mask_row_table() -> np.ndarray:
    full = _full_mask()
    return full[::_PS, :].astype(np.float32)


_MASK_ROWS = _mask_row_table()


def _sc_body(x_hbm, mrow_hbm, out_hbm, buf, mtab):
    wid = lax.axis_index("s") * 2 + lax.axis_index("c")
    base = wid * _RPW
    pltpu.sync_copy(mrow_hbm, mtab)

    def chunk(ci, carry):
        row0 = base + ci * _CHUNK
        pltpu.sync_copy(x_hbm.at[pl.ds(row0, _CHUNK)], buf)
        pr = lax.rem(ci, _HP)
        mvals = [mtab[pr, pl.ds(c * _L, _L)] for c in range(_W // _L)]
        for r in range(_CHUNK):
            for c in range(_W // _L):
                sl = pl.ds(c * _L, _L)
                buf[r, sl] = jnp.where(mvals[c] != 0.0, _MASK_VALUE, buf[r, sl])
        pltpu.sync_copy(buf, out_hbm.at[pl.ds(row0, _CHUNK)])
        return carry

    lax.fori_loop(0, _NCHUNK, chunk, 0)


def _kernel_sc(x):
    B, C, H, W = x.shape
    xr = x.reshape(B * C * H, W)
    mrows = jnp.asarray(_MASK_ROWS)
    mesh = plsc.VectorSubcoreMesh(core_axis_name="c", subcore_axis_name="s")
    run = functools.partial(
        pl.kernel,
        mesh=mesh,
        out_type=jax.ShapeDtypeStruct((B * C * H, W), x.dtype),
        scratch_types=[
            pltpu.VMEM((_CHUNK, W), jnp.float32),
            pltpu.VMEM((_HP, W), jnp.float32),
        ],
    )(_sc_body)
    out = run(xr, mrows)
    return out.reshape(B, C, H, W)


def _tc_rows(xr, mask, out_rows):
    return pl.pallas_call(
        _select_body,
        grid=(out_rows // _BLOCK_ROWS,),
        in_specs=[
            pl.BlockSpec((_BLOCK_ROWS, _W), lambda i: (0, 0)),
            pl.BlockSpec((_BLOCK_ROWS, _W), lambda i: (i, 0)),
        ],
        out_specs=pl.BlockSpec((_BLOCK_ROWS, _W), lambda i: (i, 0)),
        out_shape=jax.ShapeDtypeStruct((out_rows, _W), xr.dtype),
        compiler_params=pltpu.CompilerParams(
            dimension_semantics=("parallel",),
        ),
    )(mask, xr)


def _sc_body_rows(nchunk, in_base):
    def body(x_hbm, mrow_hbm, out_hbm, buf, mtab):
        wid = lax.axis_index("s") * 2 + lax.axis_index("c")
        base = wid * nchunk * _CHUNK
        pltpu.sync_copy(mrow_hbm, mtab)

        def chunk(ci, carry):
            row0 = base + ci * _CHUNK
            pltpu.sync_copy(x_hbm.at[pl.ds(in_base + row0, _CHUNK)], buf)
            pr = lax.rem(ci, _HP)
            mvals = [mtab[pr, pl.ds(c * _L, _L)] for c in range(_W // _L)]
            for r in range(_CHUNK):
                for c in range(_W // _L):
                    sl = pl.ds(c * _L, _L)
                    buf[r, sl] = jnp.where(
                        mvals[c] != 0.0, _MASK_VALUE, buf[r, sl])
            pltpu.sync_copy(buf, out_hbm.at[pl.ds(row0, _CHUNK)])
            return carry

        lax.fori_loop(0, nchunk, chunk, 0)

    return body


def _sc_rows(xr, mrows, out_rows, in_base):
    nchunk = out_rows // (_NW * _CHUNK)
    mesh = plsc.VectorSubcoreMesh(core_axis_name="c", subcore_axis_name="s")
    run = functools.partial(
        pl.kernel,
        mesh=mesh,
        out_type=jax.ShapeDtypeStruct((out_rows, _W), xr.dtype),
        scratch_types=[
            pltpu.VMEM((_CHUNK, _W), jnp.float32),
            pltpu.VMEM((_HP, _W), jnp.float32),
        ],
    )(_sc_body_rows(nchunk, in_base))
    return run(xr, mrows)


_SC_PLANES = 48


def _kernel_hybrid(x):
    B, C, H, W = x.shape
    rows = B * C * H
    xr = x.reshape(rows, W)
    split = rows - _SC_PLANES * _H
    mask = jnp.asarray(_MASK_BLOCK)
    mrows = jnp.asarray(_MASK_ROWS)
    top = _tc_rows(xr, mask, split)
    bot = _sc_rows(xr, mrows, rows - split, split)
    out = jnp.concatenate([top, bot], axis=0)
    return out.reshape(B, C, H, W)


kernel = _kernel_hybrid
